# baseline (speedup 1.0000x reference)
import functools

import jax
import jax.numpy as jnp
from jax import lax
from jax.experimental import pallas as pl
from jax.experimental.pallas import tpu as pltpu
from jax.experimental.pallas import tpu_sc as plsc

_ROW = 128
_G = 4


def kernel(word, positive, negatives, w_input, w_output):
    B, N = negatives.shape
    V, D = w_input.shape

    info = plsc.get_sparse_core_info()
    nc, ns = info.num_cores, info.num_subcores
    nw = nc * ns

    chunk = _ROW * _G

    word2 = word.astype(jnp.int32).reshape(B // _ROW, _ROW)
    pos2 = positive.astype(jnp.int32).reshape(B // _ROW, _ROW)
    neg2 = negatives.astype(jnp.int32).reshape(B * N // _ROW, _ROW)

    mesh = plsc.VectorSubcoreMesh(core_axis_name="c", subcore_axis_name="s")

    @functools.partial(
        pl.kernel,
        mesh=mesh,
        out_type=(
            jax.ShapeDtypeStruct((B, D), jnp.float32),
            jax.ShapeDtypeStruct((B, D), jnp.float32),
            jax.ShapeDtypeStruct((B * N, D), jnp.float32),
        ),
        scratch_types=[
            pltpu.VMEM((_G, _ROW), jnp.int32),
            pltpu.VMEM((chunk, D), jnp.float32),
            pltpu.SemaphoreType.DMA,
        ],
        compiler_params=pltpu.CompilerParams(use_tc_tiling_on_sc=False),
    )
    def gather_all(word_h, pos_h, neg_h, win_h, wout_h, ow_h, op_h, on_h,
                   idx_v, rows_v, sem):
        wid = lax.axis_index("s") * nc + lax.axis_index("c")

        def run_segment(idx_h, table_h, out_h, n_idx_rows):
            rows_per_worker = n_idx_rows // nw
            chunks = rows_per_worker // _G
            base_row = wid * rows_per_worker

            def body(i, carry):
                roff = base_row + i * _G
                pltpu.sync_copy(idx_h.at[pl.ds(roff, _G)], idx_v)
                copies = [
                    pltpu.async_copy(
                        table_h.at[idx_v.at[j]],
                        rows_v.at[pl.ds(j * _ROW, _ROW)],
                        sem,
                    )
                    for j in range(_G)
                ]
                for cp in copies:
                    cp.wait()
                pltpu.sync_copy(rows_v, out_h.at[pl.ds(roff * _ROW, chunk)])
                return carry

            lax.fori_loop(0, chunks, body, 0)

        run_segment(word_h, win_h, ow_h, B // _ROW)
        run_segment(pos_h, wout_h, op_h, B // _ROW)
        run_segment(neg_h, wout_h, on_h, B * N // _ROW)

    ow, op, on = gather_all(word2, pos2, neg2, w_input, w_output)
    return ow, op, on.reshape(B, N, D)
